# baseline (speedup 1.0000x reference)
import jax
import jax.numpy as jnp
from jax import lax
from jax.experimental import pallas as pl
from jax.experimental.pallas import tpu as pltpu
from jax.experimental.pallas import tpu_sc as plsc

VOCAB = 1000000
BATCH = 4096
HIST = 200
DIM = 64
LABELS = 10
PDIM = 16

NUM_CORES = 2
NUM_SUBCORES = 16
NUM_WORKERS = NUM_CORES * NUM_SUBCORES

PACKED_ROWS = 131072
LIN_ROWS = PACKED_ROWS * 8


_PT_BM = 8000


def _proj_body(t_ref, w_ref, o_ref):
    x = t_ref[...]
    w = w_ref[...]
    o_ref[...] = jnp.concatenate(
        [jnp.dot(x[j * (_PT_BM // 8):(j + 1) * (_PT_BM // 8), :], w,
                 preferred_element_type=jnp.float32) for j in range(8)],
        axis=1)


_proj = pl.pallas_call(
    _proj_body,
    grid=(VOCAB // _PT_BM,),
    in_specs=[
        pl.BlockSpec((_PT_BM, DIM), lambda i: (i, 0)),
        pl.BlockSpec((DIM, PDIM), lambda i: (0, 0)),
    ],
    out_specs=pl.BlockSpec((_PT_BM // 8, 128), lambda i: (i, 0)),
    out_shape=jax.ShapeDtypeStruct((PACKED_ROWS, 128), jnp.float32),
)


_RP_CH = 128
_RP_PER_W = PACKED_ROWS // NUM_WORKERS
_RP_CHUNKS = _RP_PER_W // _RP_CH


def _repack_body(src_hbm, dst_hbm, in0, in1, out0, out1, si0, si1, so0, so1):
    c = lax.axis_index("c")
    s = lax.axis_index("s")
    wid = s * NUM_CORES + c
    base = wid * _RP_PER_W

    ins, outs, sis, sos = (in0, in1), (out0, out1), (si0, si1), (so0, so1)

    pltpu.async_copy(src_hbm.at[pl.ds(base, _RP_CH)], in0, si0)
    pltpu.async_copy(src_hbm.at[pl.ds(base + _RP_CH, _RP_CH)], in1, si1)

    def pair(t, _):
        for par in range(2):
            i = 2 * t + par
            a = base + i * _RP_CH
            ib, ob, si, so = ins[par], outs[par], sis[par], sos[par]
            pltpu.make_async_copy(src_hbm.at[pl.ds(a, _RP_CH)], ib, si).wait()

            @pl.when(i >= 2)
            def _():
                pltpu.make_async_copy(
                    ob, dst_hbm.at[pl.ds((a - 2 * _RP_CH) * 8, _RP_CH * 8)],
                    so).wait()

            def vcopy(p, _):
                for j in range(8):
                    ob[8 * p + j, pl.ds(0, PDIM)] = ib[p, pl.ds(PDIM * j,
                                                                PDIM)]
                return ()

            lax.fori_loop(0, _RP_CH, vcopy, ())
            pltpu.async_copy(ob, dst_hbm.at[pl.ds(a * 8, _RP_CH * 8)], so)

            @pl.when(i + 2 < _RP_CHUNKS)
            def _():
                pltpu.async_copy(
                    src_hbm.at[pl.ds(a + 2 * _RP_CH, _RP_CH)], ib, si)
        return ()

    lax.fori_loop(0, _RP_CHUNKS // 2, pair, ())
    for k in (_RP_CHUNKS - 2, _RP_CHUNKS - 1):
        a = base + k * _RP_CH
        pltpu.make_async_copy(
            outs[k % 2], dst_hbm.at[pl.ds(a * 8, _RP_CH * 8)],
            sos[k % 2]).wait()


_repack = pl.kernel(
    _repack_body,
    out_type=jax.ShapeDtypeStruct((LIN_ROWS, PDIM), jnp.float32),
    mesh=plsc.VectorSubcoreMesh(core_axis_name="c", subcore_axis_name="s",
                                num_cores=NUM_CORES,
                                num_subcores=NUM_SUBCORES),
    scratch_types=[
        pltpu.VMEM((_RP_CH, 128), jnp.float32),
        pltpu.VMEM((_RP_CH, 128), jnp.float32),
        pltpu.VMEM((_RP_CH * 8, PDIM), jnp.float32),
        pltpu.VMEM((_RP_CH * 8, PDIM), jnp.float32),
        pltpu.SemaphoreType.DMA,
        pltpu.SemaphoreType.DMA,
        pltpu.SemaphoreType.DMA,
        pltpu.SemaphoreType.DMA,
    ],
    compiler_params=pltpu.CompilerParams(use_tc_tiling_on_sc=False),
)


B_PER_W = BATCH // NUM_WORKERS
CHUNK_A = 104
CHUNK_B = HIST - CHUNK_A


def _accum(buf, n, acc):
    def body(r, acc):
        return acc + buf[r, pl.ds(0, PDIM)]
    return lax.fori_loop(0, n, body, acc, unroll=8)


def _sc_mean_body(content_hbm, ptab_hbm, bias_hbm, means_hbm, idx_v, buf_a,
                  buf_b, out_v, bias_v, sem_a, sem_b):
    c = lax.axis_index("c")
    s = lax.axis_index("s")
    wid = s * NUM_CORES + c
    base = wid * B_PER_W

    pltpu.sync_copy(content_hbm.at[pl.ds(base, B_PER_W)], idx_v)
    pltpu.sync_copy(bias_hbm, bias_v)
    bias = bias_v[...]

    pltpu.async_copy(ptab_hbm.at[idx_v.at[0, pl.ds(0, CHUNK_A)]], buf_a, sem_a)

    def row(b, _):
        pltpu.async_copy(
            ptab_hbm.at[idx_v.at[b, pl.ds(CHUNK_A, CHUNK_B)]], buf_b, sem_b)
        pltpu.make_async_copy(
            ptab_hbm.at[idx_v.at[b, pl.ds(0, CHUNK_A)]], buf_a, sem_a).wait()
        acc = _accum(buf_a, CHUNK_A, jnp.zeros((PDIM,), jnp.float32))

        @pl.when(b + 1 < B_PER_W)
        def _():
            pltpu.async_copy(
                ptab_hbm.at[idx_v.at[b + 1, pl.ds(0, CHUNK_A)]], buf_a, sem_a)

        pltpu.make_async_copy(
            ptab_hbm.at[idx_v.at[b, pl.ds(CHUNK_A, CHUNK_B)]], buf_b,
            sem_b).wait()
        acc = _accum(buf_b, CHUNK_B, acc)
        out_v[b, pl.ds(0, PDIM)] = acc * (1.0 / HIST) + bias
        return ()

    lax.fori_loop(0, B_PER_W, row, ())
    pltpu.sync_copy(out_v, means_hbm.at[pl.ds(base, B_PER_W)])


_sc_mean = pl.kernel(
    _sc_mean_body,
    out_type=jax.ShapeDtypeStruct((BATCH, PDIM), jnp.float32),
    mesh=plsc.VectorSubcoreMesh(core_axis_name="c", subcore_axis_name="s",
                                num_cores=NUM_CORES,
                                num_subcores=NUM_SUBCORES),
    scratch_types=[
        pltpu.VMEM((B_PER_W, HIST), jnp.int32),
        pltpu.VMEM((CHUNK_A, PDIM), jnp.float32),
        pltpu.VMEM((CHUNK_B, PDIM), jnp.float32),
        pltpu.VMEM((B_PER_W, PDIM), jnp.float32),
        pltpu.VMEM((PDIM,), jnp.float32),
        pltpu.SemaphoreType.DMA,
        pltpu.SemaphoreType.DMA,
    ],
    compiler_params=pltpu.CompilerParams(use_tc_tiling_on_sc=False),
)


def kernel(content, emb_table, fc_w, fc_b):
    w_pad = jnp.zeros((DIM, PDIM), jnp.float32).at[:, :LABELS].set(fc_w.T)
    b_pad = jnp.zeros((PDIM,), jnp.float32).at[:LABELS].set(fc_b)
    ptab128 = _proj(emb_table, w_pad)
    ptab = _repack(ptab128)
    r = content.astype(jnp.int32)
    sub = r % _PT_BM
    g = 8 * (1000 * (r // _PT_BM) + sub % 1000) + sub // 1000
    means = _sc_mean(g, ptab, b_pad)
    return means[:, :LABELS]


# speedup vs baseline: 1.6619x; 1.6619x over previous
import jax
import jax.numpy as jnp
from jax import lax
from jax.experimental import pallas as pl
from jax.experimental.pallas import tpu as pltpu
from jax.experimental.pallas import tpu_sc as plsc

VOCAB = 1000000
BATCH = 4096
HIST = 200
DIM = 64
LABELS = 10
PDIM = 16

NUM_CORES = 2
NUM_SUBCORES = 16
NUM_WORKERS = NUM_CORES * NUM_SUBCORES

PACKED_ROWS = 131072
LIN_ROWS = PACKED_ROWS * 8


_PT_BM = 8192


def _proj_body(t_ref, w_ref, o_ref):
    x = t_ref[...]
    w = w_ref[...]
    dn = (((0,), (0,)), ((), ()))
    o_ref[...] = jnp.concatenate(
        [lax.dot_general(x[:, j * (_PT_BM // 8):(j + 1) * (_PT_BM // 8)], w,
                         dn, preferred_element_type=jnp.float32)
         for j in range(8)],
        axis=1)


_proj = pl.pallas_call(
    _proj_body,
    grid=((VOCAB + _PT_BM - 1) // _PT_BM,),
    in_specs=[
        pl.BlockSpec((DIM, _PT_BM), lambda i: (0, i)),
        pl.BlockSpec((DIM, PDIM), lambda i: (0, 0)),
    ],
    out_specs=pl.BlockSpec((_PT_BM // 8, 128), lambda i: (i, 0)),
    out_shape=jax.ShapeDtypeStruct((PACKED_ROWS, 128), jnp.float32),
)


_RP_CH = 128
_RP_PER_W = PACKED_ROWS // NUM_WORKERS
_RP_CHUNKS = _RP_PER_W // _RP_CH


def _repack_body(src_hbm, dst_hbm, in0, in1, out0, out1, si0, si1, so0, so1):
    c = lax.axis_index("c")
    s = lax.axis_index("s")
    wid = s * NUM_CORES + c
    base = wid * _RP_PER_W

    ins, outs, sis, sos = (in0, in1), (out0, out1), (si0, si1), (so0, so1)

    pltpu.async_copy(src_hbm.at[pl.ds(base, _RP_CH)], in0, si0)
    pltpu.async_copy(src_hbm.at[pl.ds(base + _RP_CH, _RP_CH)], in1, si1)

    def pair(t, _):
        for par in range(2):
            i = 2 * t + par
            a = base + i * _RP_CH
            ib, ob, si, so = ins[par], outs[par], sis[par], sos[par]
            pltpu.make_async_copy(src_hbm.at[pl.ds(a, _RP_CH)], ib, si).wait()

            @pl.when(i >= 2)
            def _():
                pltpu.make_async_copy(
                    ob, dst_hbm.at[pl.ds((a - 2 * _RP_CH) * 8, _RP_CH * 8)],
                    so).wait()

            def vcopy(p, _):
                for j in range(8):
                    ob[8 * p + j, pl.ds(0, PDIM)] = ib[p, pl.ds(PDIM * j,
                                                                PDIM)]
                return ()

            lax.fori_loop(0, _RP_CH, vcopy, ())
            pltpu.async_copy(ob, dst_hbm.at[pl.ds(a * 8, _RP_CH * 8)], so)

            @pl.when(i + 2 < _RP_CHUNKS)
            def _():
                pltpu.async_copy(
                    src_hbm.at[pl.ds(a + 2 * _RP_CH, _RP_CH)], ib, si)
        return ()

    lax.fori_loop(0, _RP_CHUNKS // 2, pair, ())
    for k in (_RP_CHUNKS - 2, _RP_CHUNKS - 1):
        a = base + k * _RP_CH
        pltpu.make_async_copy(
            outs[k % 2], dst_hbm.at[pl.ds(a * 8, _RP_CH * 8)],
            sos[k % 2]).wait()


_repack = pl.kernel(
    _repack_body,
    out_type=jax.ShapeDtypeStruct((LIN_ROWS, PDIM), jnp.float32),
    mesh=plsc.VectorSubcoreMesh(core_axis_name="c", subcore_axis_name="s",
                                num_cores=NUM_CORES,
                                num_subcores=NUM_SUBCORES),
    scratch_types=[
        pltpu.VMEM((_RP_CH, 128), jnp.float32),
        pltpu.VMEM((_RP_CH, 128), jnp.float32),
        pltpu.VMEM((_RP_CH * 8, PDIM), jnp.float32),
        pltpu.VMEM((_RP_CH * 8, PDIM), jnp.float32),
        pltpu.SemaphoreType.DMA,
        pltpu.SemaphoreType.DMA,
        pltpu.SemaphoreType.DMA,
        pltpu.SemaphoreType.DMA,
    ],
    compiler_params=pltpu.CompilerParams(use_tc_tiling_on_sc=False),
)


B_PER_W = BATCH // NUM_WORKERS
CHUNK_A = 104
CHUNK_B = HIST - CHUNK_A


def _accum(buf, n, acc):
    def body(r, acc):
        return acc + buf[r, pl.ds(0, PDIM)]
    return lax.fori_loop(0, n, body, acc, unroll=8)


def _sc_mean_body(content_hbm, ptab_hbm, bias_hbm, means_hbm, idx_v, buf_a,
                  buf_b, out_v, bias_v, sem_a, sem_b):
    c = lax.axis_index("c")
    s = lax.axis_index("s")
    wid = s * NUM_CORES + c
    base = wid * B_PER_W

    pltpu.sync_copy(content_hbm.at[pl.ds(base, B_PER_W)], idx_v)
    pltpu.sync_copy(bias_hbm, bias_v)
    bias = bias_v[...]

    pltpu.async_copy(ptab_hbm.at[idx_v.at[0, pl.ds(0, CHUNK_A)]], buf_a, sem_a)

    def row(b, _):
        pltpu.async_copy(
            ptab_hbm.at[idx_v.at[b, pl.ds(CHUNK_A, CHUNK_B)]], buf_b, sem_b)
        pltpu.make_async_copy(
            ptab_hbm.at[idx_v.at[b, pl.ds(0, CHUNK_A)]], buf_a, sem_a).wait()
        acc = _accum(buf_a, CHUNK_A, jnp.zeros((PDIM,), jnp.float32))

        @pl.when(b + 1 < B_PER_W)
        def _():
            pltpu.async_copy(
                ptab_hbm.at[idx_v.at[b + 1, pl.ds(0, CHUNK_A)]], buf_a, sem_a)

        pltpu.make_async_copy(
            ptab_hbm.at[idx_v.at[b, pl.ds(CHUNK_A, CHUNK_B)]], buf_b,
            sem_b).wait()
        acc = _accum(buf_b, CHUNK_B, acc)
        out_v[b, pl.ds(0, PDIM)] = acc * (1.0 / HIST) + bias
        return ()

    lax.fori_loop(0, B_PER_W, row, ())
    pltpu.sync_copy(out_v, means_hbm.at[pl.ds(base, B_PER_W)])


_sc_mean = pl.kernel(
    _sc_mean_body,
    out_type=jax.ShapeDtypeStruct((BATCH, PDIM), jnp.float32),
    mesh=plsc.VectorSubcoreMesh(core_axis_name="c", subcore_axis_name="s",
                                num_cores=NUM_CORES,
                                num_subcores=NUM_SUBCORES),
    scratch_types=[
        pltpu.VMEM((B_PER_W, HIST), jnp.int32),
        pltpu.VMEM((CHUNK_A, PDIM), jnp.float32),
        pltpu.VMEM((CHUNK_B, PDIM), jnp.float32),
        pltpu.VMEM((B_PER_W, PDIM), jnp.float32),
        pltpu.VMEM((PDIM,), jnp.float32),
        pltpu.SemaphoreType.DMA,
        pltpu.SemaphoreType.DMA,
    ],
    compiler_params=pltpu.CompilerParams(use_tc_tiling_on_sc=False),
)


def kernel(content, emb_table, fc_w, fc_b):
    w_pad = jnp.zeros((DIM, PDIM), jnp.float32).at[:, :LABELS].set(fc_w.T)
    b_pad = jnp.zeros((PDIM,), jnp.float32).at[:LABELS].set(fc_b)
    ptab128 = _proj(emb_table.T, w_pad)
    ptab = _repack(ptab128)
    r = content.astype(jnp.int32)
    sub = r % _PT_BM
    q = _PT_BM // 8
    g = 8 * (q * (r // _PT_BM) + sub % q) + sub // q
    means = _sc_mean(g, ptab, b_pad)
    return means[:, :LABELS]


# speedup vs baseline: 1.6768x; 1.0090x over previous
import jax
import jax.numpy as jnp
from jax import lax
from jax.experimental import pallas as pl
from jax.experimental.pallas import tpu as pltpu
from jax.experimental.pallas import tpu_sc as plsc

VOCAB = 1000000
BATCH = 4096
HIST = 200
DIM = 64
LABELS = 10
PDIM = 16

NUM_CORES = 2
NUM_SUBCORES = 16
NUM_WORKERS = NUM_CORES * NUM_SUBCORES

PACKED_ROWS = 131072
LIN_ROWS = PACKED_ROWS * 8


_PT_BM = 16384


def _proj_body(t_ref, w_ref, o_ref):
    x = t_ref[...]
    w = w_ref[...]
    dn = (((0,), (0,)), ((), ()))
    o_ref[...] = jnp.concatenate(
        [lax.dot_general(x[:, j * (_PT_BM // 8):(j + 1) * (_PT_BM // 8)], w,
                         dn, preferred_element_type=jnp.float32)
         for j in range(8)],
        axis=1)


_proj = pl.pallas_call(
    _proj_body,
    grid=((VOCAB + _PT_BM - 1) // _PT_BM,),
    in_specs=[
        pl.BlockSpec((DIM, _PT_BM), lambda i: (0, i)),
        pl.BlockSpec((DIM, PDIM), lambda i: (0, 0)),
    ],
    out_specs=pl.BlockSpec((_PT_BM // 8, 128), lambda i: (i, 0)),
    out_shape=jax.ShapeDtypeStruct((PACKED_ROWS, 128), jnp.float32),
)


_RP_CH = 128
_RP_PER_W = PACKED_ROWS // NUM_WORKERS
_RP_CHUNKS = _RP_PER_W // _RP_CH


def _repack_body(src_hbm, dst_hbm, in0, in1, out0, out1, si0, si1, so0, so1):
    c = lax.axis_index("c")
    s = lax.axis_index("s")
    wid = s * NUM_CORES + c
    base = wid * _RP_PER_W

    ins, outs, sis, sos = (in0, in1), (out0, out1), (si0, si1), (so0, so1)

    pltpu.async_copy(src_hbm.at[pl.ds(base, _RP_CH)], in0, si0)
    pltpu.async_copy(src_hbm.at[pl.ds(base + _RP_CH, _RP_CH)], in1, si1)

    def pair(t, _):
        for par in range(2):
            i = 2 * t + par
            a = base + i * _RP_CH
            ib, ob, si, so = ins[par], outs[par], sis[par], sos[par]
            pltpu.make_async_copy(src_hbm.at[pl.ds(a, _RP_CH)], ib, si).wait()

            @pl.when(i >= 2)
            def _():
                pltpu.make_async_copy(
                    ob, dst_hbm.at[pl.ds((a - 2 * _RP_CH) * 8, _RP_CH * 8)],
                    so).wait()

            def vcopy(p, _):
                for j in range(8):
                    ob[8 * p + j, pl.ds(0, PDIM)] = ib[p, pl.ds(PDIM * j,
                                                                PDIM)]
                return ()

            lax.fori_loop(0, _RP_CH, vcopy, ())
            pltpu.async_copy(ob, dst_hbm.at[pl.ds(a * 8, _RP_CH * 8)], so)

            @pl.when(i + 2 < _RP_CHUNKS)
            def _():
                pltpu.async_copy(
                    src_hbm.at[pl.ds(a + 2 * _RP_CH, _RP_CH)], ib, si)
        return ()

    lax.fori_loop(0, _RP_CHUNKS // 2, pair, ())
    for k in (_RP_CHUNKS - 2, _RP_CHUNKS - 1):
        a = base + k * _RP_CH
        pltpu.make_async_copy(
            outs[k % 2], dst_hbm.at[pl.ds(a * 8, _RP_CH * 8)],
            sos[k % 2]).wait()


_repack = pl.kernel(
    _repack_body,
    out_type=jax.ShapeDtypeStruct((LIN_ROWS, PDIM), jnp.float32),
    mesh=plsc.VectorSubcoreMesh(core_axis_name="c", subcore_axis_name="s",
                                num_cores=NUM_CORES,
                                num_subcores=NUM_SUBCORES),
    scratch_types=[
        pltpu.VMEM((_RP_CH, 128), jnp.float32),
        pltpu.VMEM((_RP_CH, 128), jnp.float32),
        pltpu.VMEM((_RP_CH * 8, PDIM), jnp.float32),
        pltpu.VMEM((_RP_CH * 8, PDIM), jnp.float32),
        pltpu.SemaphoreType.DMA,
        pltpu.SemaphoreType.DMA,
        pltpu.SemaphoreType.DMA,
        pltpu.SemaphoreType.DMA,
    ],
    compiler_params=pltpu.CompilerParams(use_tc_tiling_on_sc=False),
)


B_PER_W = BATCH // NUM_WORKERS
CHUNK_A = 104
CHUNK_B = HIST - CHUNK_A


def _accum(buf, n, acc):
    def body(r, acc):
        return acc + buf[r, pl.ds(0, PDIM)]
    return lax.fori_loop(0, n, body, acc, unroll=8)


def _sc_mean_body(content_hbm, ptab_hbm, bias_hbm, means_hbm, idx_v, buf_a,
                  buf_b, out_v, bias_v, sem_a, sem_b):
    c = lax.axis_index("c")
    s = lax.axis_index("s")
    wid = s * NUM_CORES + c
    base = wid * B_PER_W

    pltpu.sync_copy(content_hbm.at[pl.ds(base, B_PER_W)], idx_v)
    pltpu.sync_copy(bias_hbm, bias_v)
    bias = bias_v[...]

    pltpu.async_copy(ptab_hbm.at[idx_v.at[0, pl.ds(0, CHUNK_A)]], buf_a, sem_a)

    def row(b, _):
        pltpu.async_copy(
            ptab_hbm.at[idx_v.at[b, pl.ds(CHUNK_A, CHUNK_B)]], buf_b, sem_b)
        pltpu.make_async_copy(
            ptab_hbm.at[idx_v.at[b, pl.ds(0, CHUNK_A)]], buf_a, sem_a).wait()
        acc = _accum(buf_a, CHUNK_A, jnp.zeros((PDIM,), jnp.float32))

        @pl.when(b + 1 < B_PER_W)
        def _():
            pltpu.async_copy(
                ptab_hbm.at[idx_v.at[b + 1, pl.ds(0, CHUNK_A)]], buf_a, sem_a)

        pltpu.make_async_copy(
            ptab_hbm.at[idx_v.at[b, pl.ds(CHUNK_A, CHUNK_B)]], buf_b,
            sem_b).wait()
        acc = _accum(buf_b, CHUNK_B, acc)
        out_v[b, pl.ds(0, PDIM)] = acc * (1.0 / HIST) + bias
        return ()

    lax.fori_loop(0, B_PER_W, row, ())
    pltpu.sync_copy(out_v, means_hbm.at[pl.ds(base, B_PER_W)])


_sc_mean = pl.kernel(
    _sc_mean_body,
    out_type=jax.ShapeDtypeStruct((BATCH, PDIM), jnp.float32),
    mesh=plsc.VectorSubcoreMesh(core_axis_name="c", subcore_axis_name="s",
                                num_cores=NUM_CORES,
                                num_subcores=NUM_SUBCORES),
    scratch_types=[
        pltpu.VMEM((B_PER_W, HIST), jnp.int32),
        pltpu.VMEM((CHUNK_A, PDIM), jnp.float32),
        pltpu.VMEM((CHUNK_B, PDIM), jnp.float32),
        pltpu.VMEM((B_PER_W, PDIM), jnp.float32),
        pltpu.VMEM((PDIM,), jnp.float32),
        pltpu.SemaphoreType.DMA,
        pltpu.SemaphoreType.DMA,
    ],
    compiler_params=pltpu.CompilerParams(use_tc_tiling_on_sc=False),
)


def kernel(content, emb_table, fc_w, fc_b):
    w_pad = jnp.zeros((DIM, PDIM), jnp.float32).at[:, :LABELS].set(fc_w.T)
    b_pad = jnp.zeros((PDIM,), jnp.float32).at[:LABELS].set(fc_b)
    ptab128 = _proj(emb_table.T, w_pad)
    ptab = _repack(ptab128)
    r = content.astype(jnp.int32)
    sub = r % _PT_BM
    q = _PT_BM // 8
    g = 8 * (q * (r // _PT_BM) + sub % q) + sub // q
    means = _sc_mean(g, ptab, b_pad)
    return means[:, :LABELS]


# speedup vs baseline: 1.8919x; 1.1283x over previous
import jax
import jax.numpy as jnp
from jax import lax
from jax.experimental import pallas as pl
from jax.experimental.pallas import tpu as pltpu
from jax.experimental.pallas import tpu_sc as plsc

VOCAB = 1000000
BATCH = 4096
HIST = 200
DIM = 64
LABELS = 10
PDIM = 16

NUM_CORES = 2
NUM_SUBCORES = 16
NUM_WORKERS = NUM_CORES * NUM_SUBCORES

PACKED_ROWS = 131072
LIN_ROWS = PACKED_ROWS * 8


_PT_BM = 16384


def _proj_body(t_ref, w_ref, o_ref):
    x = t_ref[...]
    w = w_ref[...]
    dn = (((0,), (0,)), ((), ()))
    o_ref[...] = jnp.concatenate(
        [lax.dot_general(x[:, j * (_PT_BM // 8):(j + 1) * (_PT_BM // 8)], w,
                         dn, preferred_element_type=jnp.float32)
         for j in range(8)],
        axis=1)


_proj = pl.pallas_call(
    _proj_body,
    grid=((VOCAB + _PT_BM - 1) // _PT_BM,),
    in_specs=[
        pl.BlockSpec((DIM, _PT_BM), lambda i: (0, i)),
        pl.BlockSpec((DIM, PDIM), lambda i: (0, 0)),
    ],
    out_specs=pl.BlockSpec((_PT_BM // 8, 128), lambda i: (i, 0)),
    out_shape=jax.ShapeDtypeStruct((PACKED_ROWS, 128), jnp.float32),
)


_RP_CH = 128
_RP_PER_W = PACKED_ROWS // NUM_WORKERS
_RP_CHUNKS = _RP_PER_W // _RP_CH


def _repack_body(src_hbm, dst_hbm, in0, in1, out0, out1, si0, si1, so0, so1):
    c = lax.axis_index("c")
    s = lax.axis_index("s")
    wid = s * NUM_CORES + c
    base = wid * _RP_PER_W

    ins, outs, sis, sos = (in0, in1), (out0, out1), (si0, si1), (so0, so1)

    pltpu.async_copy(src_hbm.at[pl.ds(base, _RP_CH)], in0, si0)
    pltpu.async_copy(src_hbm.at[pl.ds(base + _RP_CH, _RP_CH)], in1, si1)

    def pair(t, _):
        for par in range(2):
            i = 2 * t + par
            a = base + i * _RP_CH
            ib, ob, si, so = ins[par], outs[par], sis[par], sos[par]
            pltpu.make_async_copy(src_hbm.at[pl.ds(a, _RP_CH)], ib, si).wait()

            @pl.when(i >= 2)
            def _():
                pltpu.make_async_copy(
                    ob, dst_hbm.at[pl.ds((a - 2 * _RP_CH) * 8, _RP_CH * 8)],
                    so).wait()

            def vcopy(p, _):
                for j in range(8):
                    ob[8 * p + j, pl.ds(0, PDIM)] = ib[p, pl.ds(PDIM * j,
                                                                PDIM)]
                return ()

            lax.fori_loop(0, _RP_CH, vcopy, ())
            pltpu.async_copy(ob, dst_hbm.at[pl.ds(a * 8, _RP_CH * 8)], so)

            @pl.when(i + 2 < _RP_CHUNKS)
            def _():
                pltpu.async_copy(
                    src_hbm.at[pl.ds(a + 2 * _RP_CH, _RP_CH)], ib, si)
        return ()

    lax.fori_loop(0, _RP_CHUNKS // 2, pair, ())
    for k in (_RP_CHUNKS - 2, _RP_CHUNKS - 1):
        a = base + k * _RP_CH
        pltpu.make_async_copy(
            outs[k % 2], dst_hbm.at[pl.ds(a * 8, _RP_CH * 8)],
            sos[k % 2]).wait()


_repack = pl.kernel(
    _repack_body,
    out_type=jax.ShapeDtypeStruct((LIN_ROWS, PDIM), jnp.float32),
    mesh=plsc.VectorSubcoreMesh(core_axis_name="c", subcore_axis_name="s",
                                num_cores=NUM_CORES,
                                num_subcores=NUM_SUBCORES),
    scratch_types=[
        pltpu.VMEM((_RP_CH, 128), jnp.float32),
        pltpu.VMEM((_RP_CH, 128), jnp.float32),
        pltpu.VMEM((_RP_CH * 8, PDIM), jnp.float32),
        pltpu.VMEM((_RP_CH * 8, PDIM), jnp.float32),
        pltpu.SemaphoreType.DMA,
        pltpu.SemaphoreType.DMA,
        pltpu.SemaphoreType.DMA,
        pltpu.SemaphoreType.DMA,
    ],
    compiler_params=pltpu.CompilerParams(use_tc_tiling_on_sc=False),
)


B_PER_W = BATCH // NUM_WORKERS
CHUNK_A = 104
CHUNK_B = HIST - CHUNK_A


def _accum(buf, n, acc):
    def body(r, acc):
        return acc + buf[r, pl.ds(0, PDIM)]
    return lax.fori_loop(0, n, body, acc, unroll=8)


def _sc_mean_body(content_hbm, ptab_hbm, bias_hbm, means_hbm, idx_v, a0, a1,
                  b0, b1, out_v, bias_v, sa0, sa1, sb0, sb1):
    c = lax.axis_index("c")
    s = lax.axis_index("s")
    wid = s * NUM_CORES + c
    base = wid * B_PER_W

    pltpu.sync_copy(content_hbm.at[pl.ds(base, B_PER_W)], idx_v)
    pltpu.sync_copy(bias_hbm, bias_v)
    bias = bias_v[...]

    bufs_a, bufs_b = (a0, a1), (b0, b1)
    sems_a, sems_b = (sa0, sa1), (sb0, sb1)

    for par in range(2):
        pltpu.async_copy(ptab_hbm.at[idx_v.at[par, pl.ds(0, CHUNK_A)]],
                         bufs_a[par], sems_a[par])
        pltpu.async_copy(ptab_hbm.at[idx_v.at[par, pl.ds(CHUNK_A, CHUNK_B)]],
                         bufs_b[par], sems_b[par])

    def pair(t, _):
        for par in range(2):
            b = 2 * t + par
            buf_a, buf_b = bufs_a[par], bufs_b[par]
            sem_a, sem_b = sems_a[par], sems_b[par]
            pltpu.make_async_copy(
                ptab_hbm.at[idx_v.at[b, pl.ds(0, CHUNK_A)]], buf_a,
                sem_a).wait()
            acc = _accum(buf_a, CHUNK_A, jnp.zeros((PDIM,), jnp.float32))

            @pl.when(b + 2 < B_PER_W)
            def _():
                pltpu.async_copy(
                    ptab_hbm.at[idx_v.at[b + 2, pl.ds(0, CHUNK_A)]], buf_a,
                    sem_a)

            pltpu.make_async_copy(
                ptab_hbm.at[idx_v.at[b, pl.ds(CHUNK_A, CHUNK_B)]], buf_b,
                sem_b).wait()
            acc = _accum(buf_b, CHUNK_B, acc)

            @pl.when(b + 2 < B_PER_W)
            def _():
                pltpu.async_copy(
                    ptab_hbm.at[idx_v.at[b + 2, pl.ds(CHUNK_A, CHUNK_B)]],
                    buf_b, sem_b)

            out_v[b, pl.ds(0, PDIM)] = acc * (1.0 / HIST) + bias
        return ()

    lax.fori_loop(0, B_PER_W // 2, pair, ())
    pltpu.sync_copy(out_v, means_hbm.at[pl.ds(base, B_PER_W)])


_sc_mean = pl.kernel(
    _sc_mean_body,
    out_type=jax.ShapeDtypeStruct((BATCH, PDIM), jnp.float32),
    mesh=plsc.VectorSubcoreMesh(core_axis_name="c", subcore_axis_name="s",
                                num_cores=NUM_CORES,
                                num_subcores=NUM_SUBCORES),
    scratch_types=[
        pltpu.VMEM((B_PER_W, HIST), jnp.int32),
        pltpu.VMEM((CHUNK_A, PDIM), jnp.float32),
        pltpu.VMEM((CHUNK_A, PDIM), jnp.float32),
        pltpu.VMEM((CHUNK_B, PDIM), jnp.float32),
        pltpu.VMEM((CHUNK_B, PDIM), jnp.float32),
        pltpu.VMEM((B_PER_W, PDIM), jnp.float32),
        pltpu.VMEM((PDIM,), jnp.float32),
        pltpu.SemaphoreType.DMA,
        pltpu.SemaphoreType.DMA,
        pltpu.SemaphoreType.DMA,
        pltpu.SemaphoreType.DMA,
    ],
    compiler_params=pltpu.CompilerParams(use_tc_tiling_on_sc=False),
)


def kernel(content, emb_table, fc_w, fc_b):
    w_pad = jnp.zeros((DIM, PDIM), jnp.float32).at[:, :LABELS].set(fc_w.T)
    b_pad = jnp.zeros((PDIM,), jnp.float32).at[:LABELS].set(fc_b)
    ptab128 = _proj(emb_table.T, w_pad)
    ptab = _repack(ptab128)
    r = content.astype(jnp.int32)
    sub = r % _PT_BM
    q = _PT_BM // 8
    g = 8 * (q * (r // _PT_BM) + sub % q) + sub // q
    means = _sc_mean(g, ptab, b_pad)
    return means[:, :LABELS]


# speedup vs baseline: 2.0084x; 1.0615x over previous
import jax
import jax.numpy as jnp
from jax import lax
from jax.experimental import pallas as pl
from jax.experimental.pallas import tpu as pltpu
from jax.experimental.pallas import tpu_sc as plsc

VOCAB = 1000000
BATCH = 4096
HIST = 200
DIM = 64
LABELS = 10
PDIM = 16

NUM_CORES = 2
NUM_SUBCORES = 16
NUM_WORKERS = NUM_CORES * NUM_SUBCORES

PACKED_ROWS = 131072
LIN_ROWS = PACKED_ROWS * 8


_PT_BM = 16384


def _proj_body(t_ref, w_ref, o_ref):
    x = t_ref[...]
    w = w_ref[...]
    dn = (((0,), (0,)), ((), ()))
    o_ref[...] = jnp.concatenate(
        [lax.dot_general(x[:, j * (_PT_BM // 8):(j + 1) * (_PT_BM // 8)], w,
                         dn, preferred_element_type=jnp.float32)
         for j in range(8)],
        axis=1)


_proj = pl.pallas_call(
    _proj_body,
    grid=((VOCAB + _PT_BM - 1) // _PT_BM,),
    in_specs=[
        pl.BlockSpec((DIM, _PT_BM), lambda i: (0, i)),
        pl.BlockSpec((DIM, PDIM), lambda i: (0, 0)),
    ],
    out_specs=pl.BlockSpec((_PT_BM // 8, 128), lambda i: (i, 0)),
    out_shape=jax.ShapeDtypeStruct((PACKED_ROWS, 128), jnp.float32),
)


_RP_CH = 128
_RP_PER_W = PACKED_ROWS // NUM_WORKERS
_RP_CHUNKS = _RP_PER_W // _RP_CH


def _repack_body(src_hbm, dst_hbm, in0, in1, out0, out1, si0, si1, so0, so1):
    c = lax.axis_index("c")
    s = lax.axis_index("s")
    wid = s * NUM_CORES + c
    base = wid * _RP_PER_W

    ins, outs, sis, sos = (in0, in1), (out0, out1), (si0, si1), (so0, so1)

    pltpu.async_copy(src_hbm.at[pl.ds(base, _RP_CH)], in0, si0)
    pltpu.async_copy(src_hbm.at[pl.ds(base + _RP_CH, _RP_CH)], in1, si1)

    def pair(t, _):
        for par in range(2):
            i = 2 * t + par
            a = base + i * _RP_CH
            ib, ob, si, so = ins[par], outs[par], sis[par], sos[par]
            pltpu.make_async_copy(src_hbm.at[pl.ds(a, _RP_CH)], ib, si).wait()

            @pl.when(i >= 2)
            def _():
                pltpu.make_async_copy(
                    ob, dst_hbm.at[pl.ds((a - 2 * _RP_CH) * 8, _RP_CH * 8)],
                    so).wait()

            def vcopy(p, _):
                for j in range(8):
                    ob[8 * p + j, pl.ds(0, PDIM)] = ib[p, pl.ds(PDIM * j,
                                                                PDIM)]
                return ()

            lax.fori_loop(0, _RP_CH, vcopy, ())
            pltpu.async_copy(ob, dst_hbm.at[pl.ds(a * 8, _RP_CH * 8)], so)

            @pl.when(i + 2 < _RP_CHUNKS)
            def _():
                pltpu.async_copy(
                    src_hbm.at[pl.ds(a + 2 * _RP_CH, _RP_CH)], ib, si)
        return ()

    lax.fori_loop(0, _RP_CHUNKS // 2, pair, ())
    for k in (_RP_CHUNKS - 2, _RP_CHUNKS - 1):
        a = base + k * _RP_CH
        pltpu.make_async_copy(
            outs[k % 2], dst_hbm.at[pl.ds(a * 8, _RP_CH * 8)],
            sos[k % 2]).wait()


_repack = pl.kernel(
    _repack_body,
    out_type=jax.ShapeDtypeStruct((LIN_ROWS, PDIM), jnp.float32),
    mesh=plsc.VectorSubcoreMesh(core_axis_name="c", subcore_axis_name="s",
                                num_cores=NUM_CORES,
                                num_subcores=NUM_SUBCORES),
    scratch_types=[
        pltpu.VMEM((_RP_CH, 128), jnp.float32),
        pltpu.VMEM((_RP_CH, 128), jnp.float32),
        pltpu.VMEM((_RP_CH * 8, PDIM), jnp.float32),
        pltpu.VMEM((_RP_CH * 8, PDIM), jnp.float32),
        pltpu.SemaphoreType.DMA,
        pltpu.SemaphoreType.DMA,
        pltpu.SemaphoreType.DMA,
        pltpu.SemaphoreType.DMA,
    ],
    compiler_params=pltpu.CompilerParams(use_tc_tiling_on_sc=False),
)


B_PER_W = BATCH // NUM_WORKERS
CHUNK_A = 104
CHUNK_B = HIST - CHUNK_A


def _accum(buf, n, acc):
    def body(r, acc):
        return acc + buf[r, pl.ds(0, PDIM)]
    return lax.fori_loop(0, n, body, acc, unroll=8)


def _sc_mean_body(content_hbm, ptab_hbm, bias_hbm, means_hbm, idx_v, a0, a1,
                  a2, a3, b0, b1, b2, b3, out_v, bias_v, sa0, sa1, sa2, sa3,
                  sb0, sb1, sb2, sb3):
    c = lax.axis_index("c")
    s = lax.axis_index("s")
    wid = s * NUM_CORES + c
    base = wid * B_PER_W

    pltpu.sync_copy(content_hbm.at[pl.ds(base, B_PER_W)], idx_v)
    pltpu.sync_copy(bias_hbm, bias_v)
    bias = bias_v[...]

    bufs_a, bufs_b = (a0, a1, a2, a3), (b0, b1, b2, b3)
    sems_a, sems_b = (sa0, sa1, sa2, sa3), (sb0, sb1, sb2, sb3)

    for par in range(4):
        pltpu.async_copy(ptab_hbm.at[idx_v.at[par, pl.ds(0, CHUNK_A)]],
                         bufs_a[par], sems_a[par])
        pltpu.async_copy(ptab_hbm.at[idx_v.at[par, pl.ds(CHUNK_A, CHUNK_B)]],
                         bufs_b[par], sems_b[par])

    def pair(t, _):
        for par in range(4):
            b = 4 * t + par
            buf_a, buf_b = bufs_a[par], bufs_b[par]
            sem_a, sem_b = sems_a[par], sems_b[par]
            pltpu.make_async_copy(
                ptab_hbm.at[idx_v.at[b, pl.ds(0, CHUNK_A)]], buf_a,
                sem_a).wait()
            acc = _accum(buf_a, CHUNK_A, jnp.zeros((PDIM,), jnp.float32))

            @pl.when(b + 4 < B_PER_W)
            def _():
                pltpu.async_copy(
                    ptab_hbm.at[idx_v.at[b + 4, pl.ds(0, CHUNK_A)]], buf_a,
                    sem_a)

            pltpu.make_async_copy(
                ptab_hbm.at[idx_v.at[b, pl.ds(CHUNK_A, CHUNK_B)]], buf_b,
                sem_b).wait()
            acc = _accum(buf_b, CHUNK_B, acc)

            @pl.when(b + 4 < B_PER_W)
            def _():
                pltpu.async_copy(
                    ptab_hbm.at[idx_v.at[b + 4, pl.ds(CHUNK_A, CHUNK_B)]],
                    buf_b, sem_b)

            out_v[b, pl.ds(0, PDIM)] = acc * (1.0 / HIST) + bias
        return ()

    lax.fori_loop(0, B_PER_W // 4, pair, ())
    pltpu.sync_copy(out_v, means_hbm.at[pl.ds(base, B_PER_W)])


_sc_mean = pl.kernel(
    _sc_mean_body,
    out_type=jax.ShapeDtypeStruct((BATCH, PDIM), jnp.float32),
    mesh=plsc.VectorSubcoreMesh(core_axis_name="c", subcore_axis_name="s",
                                num_cores=NUM_CORES,
                                num_subcores=NUM_SUBCORES),
    scratch_types=[
        pltpu.VMEM((B_PER_W, HIST), jnp.int32),
        pltpu.VMEM((CHUNK_A, PDIM), jnp.float32),
        pltpu.VMEM((CHUNK_A, PDIM), jnp.float32),
        pltpu.VMEM((CHUNK_A, PDIM), jnp.float32),
        pltpu.VMEM((CHUNK_A, PDIM), jnp.float32),
        pltpu.VMEM((CHUNK_B, PDIM), jnp.float32),
        pltpu.VMEM((CHUNK_B, PDIM), jnp.float32),
        pltpu.VMEM((CHUNK_B, PDIM), jnp.float32),
        pltpu.VMEM((CHUNK_B, PDIM), jnp.float32),
        pltpu.VMEM((B_PER_W, PDIM), jnp.float32),
        pltpu.VMEM((PDIM,), jnp.float32),
        pltpu.SemaphoreType.DMA,
        pltpu.SemaphoreType.DMA,
        pltpu.SemaphoreType.DMA,
        pltpu.SemaphoreType.DMA,
        pltpu.SemaphoreType.DMA,
        pltpu.SemaphoreType.DMA,
        pltpu.SemaphoreType.DMA,
        pltpu.SemaphoreType.DMA,
    ],
    compiler_params=pltpu.CompilerParams(use_tc_tiling_on_sc=False),
)


def kernel(content, emb_table, fc_w, fc_b):
    w_pad = jnp.zeros((DIM, PDIM), jnp.float32).at[:, :LABELS].set(fc_w.T)
    b_pad = jnp.zeros((PDIM,), jnp.float32).at[:LABELS].set(fc_b)
    ptab128 = _proj(emb_table.T, w_pad)
    ptab = _repack(ptab128)
    r = content.astype(jnp.int32)
    sub = r % _PT_BM
    q = _PT_BM // 8
    g = 8 * (q * (r // _PT_BM) + sub % q) + sub // q
    means = _sc_mean(g, ptab, b_pad)
    return means[:, :LABELS]
